# Initial kernel scaffold; baseline (speedup 1.0000x reference)
import jax
import jax.numpy as jnp
from jax.experimental import pallas as pl


def kernel(numerical_feature_batch, embedding_index_batch_list, embedding_offset_batch_list, tables, bw0, bb0, bw1, bb1, bw2, bb2, tw0, tb0, tw1, tb1, tw2, tb2):
    raise NotImplementedError("write your pallas kernel here")


# speedup vs baseline: 1.1040x; 1.1040x over previous
import functools

import jax
import jax.numpy as jnp
import numpy as np
from jax import lax
from jax.experimental import pallas as pl
from jax.experimental.pallas import tpu as pltpu
from jax.experimental.pallas import tpu_sc as plsc

_B = 4096
_DENSE = 13
_NT = 26
_VOCAB = 100000
_D = 64
_BAG = 20
_NI = _NT + 1

_NW = 32
_BAGS = _B * _NT
_BPW = _BAGS // _NW
_C = 32
_NCH = _BPW // _C
_IPC = _C * _BAG
_NG = _IPC // 128

_PAIR_ROWS = np.array([i for i in range(_NI) for _ in range(i)], dtype=np.int32)
_PAIR_COLS = np.array([j for i in range(_NI) for j in range(i)], dtype=np.int32)
_PAIR_FLAT = _PAIR_ROWS * _NI + _PAIR_COLS


def _sc_bag_sum_body(tables_hbm, idx_hbm, out_hbm, idx_v, rows_v, out_v, sem):
    w = lax.axis_index("s") * 2 + lax.axis_index("c")

    def chunk_body(c, carry):
        pltpu.sync_copy(idx_hbm.at[w, c], idx_v)
        cps = [
            pltpu.async_copy(
                tables_hbm.at[idx_v.at[j]], rows_v.at[pl.ds(j * 128, 128)], sem
            )
            for j in range(_NG)
        ]
        for cp in cps:
            cp.wait()

        def bag_body(i, carry2):
            base = i * _BAG
            for dd in range(_D // 16):
                sl = pl.ds(dd * 16, 16)
                acc = rows_v[base, sl]
                for j in range(1, _BAG):
                    acc = acc + rows_v[base + j, sl]
                out_v[i, sl] = acc
            return carry2

        lax.fori_loop(0, _C, bag_body, 0)
        pltpu.sync_copy(out_v, out_hbm.at[pl.ds(w * _BPW + c * _C, _C)])
        return carry

    lax.fori_loop(0, _NCH, chunk_body, 0)


@functools.cache
def _bag_sum_kernel():
    return pl.kernel(
        _sc_bag_sum_body,
        out_type=jax.ShapeDtypeStruct((_BAGS, _D), jnp.float32),
        mesh=plsc.VectorSubcoreMesh(core_axis_name="c", subcore_axis_name="s"),
        scratch_types=[
            pltpu.VMEM((_NG, 128), jnp.int32),
            pltpu.VMEM((_IPC, _D), jnp.float32),
            pltpu.VMEM((_C, _D), jnp.float32),
            pltpu.SemaphoreType.DMA,
        ],
        compiler_params=pltpu.CompilerParams(use_tc_tiling_on_sc=False),
    )


def _dense_body(num_ref, emb_ref, bw0_ref, bb0_ref, bw1_ref, bb1_ref, bw2_ref,
                bb2_ref, tw0x_ref, w729_ref, tb0_ref, tw1_ref, tb1_ref,
                tw2_ref, tb2_ref, out_ref):
    f32 = jnp.float32
    x = num_ref[...]
    x = jnp.maximum(jnp.dot(x, bw0_ref[...], preferred_element_type=f32) + bb0_ref[...], 0.0)
    x = jnp.maximum(jnp.dot(x, bw1_ref[...], preferred_element_type=f32) + bb1_ref[...], 0.0)
    x = jnp.maximum(jnp.dot(x, bw2_ref[...], preferred_element_type=f32) + bb2_ref[...], 0.0)
    a = jnp.concatenate([x[:, None, :], emb_ref[...]], axis=1)
    zs = []
    for n in range(_NI):
        zs.append(jnp.sum(a * a[:, n : n + 1, :], axis=-1))
    z = jnp.concatenate(zs, axis=-1)
    t = jnp.dot(x, tw0x_ref[...], preferred_element_type=f32)
    t = t + jnp.dot(z, w729_ref[...], preferred_element_type=f32)
    t = jnp.maximum(t + tb0_ref[...], 0.0)
    t = jnp.maximum(jnp.dot(t, tw1_ref[...], preferred_element_type=f32) + tb1_ref[...], 0.0)
    out_ref[...] = jnp.maximum(
        jnp.dot(t, tw2_ref[...], preferred_element_type=f32) + tb2_ref[...], 0.0
    )


def _dense_forward(num, emb, bw0, bb0, bw1, bb1, bw2, bb2, tw0x, w729, tb0,
                   tw1, tb1, tw2, tb2):
    bb = 256
    grid = (_B // bb,)

    def row_spec(shape):
        return pl.BlockSpec((bb,) + shape[1:], lambda i: (i,) + (0,) * (len(shape) - 1))

    def full_spec(shape):
        return pl.BlockSpec(shape, lambda i: (0,) * len(shape))

    ws = [bw0, bb0, bw1, bb1, bw2, bb2, tw0x, w729, tb0, tw1, tb1, tw2, tb2]
    return pl.pallas_call(
        _dense_body,
        grid=grid,
        in_specs=[row_spec(num.shape), row_spec(emb.shape)] + [full_spec(w.shape) for w in ws],
        out_specs=row_spec((_B, 1)),
        out_shape=jax.ShapeDtypeStruct((_B, 1), jnp.float32),
    )(num, emb, *ws)


def kernel(numerical_feature_batch, embedding_index_batch_list,
           embedding_offset_batch_list, tables, bw0, bb0, bw1, bb1, bw2, bb2,
           tw0, tb0, tw1, tb1, tw2, tb2):
    del embedding_offset_batch_list
    tabs = tables.reshape(_NT * _VOCAB, _D)
    idx_bt = jnp.transpose(embedding_index_batch_list, (1, 0, 2))
    idx_bt = idx_bt + (jnp.arange(_NT, dtype=jnp.int32) * _VOCAB)[None, :, None]
    idx_r = idx_bt.reshape(_NW, _NCH, _NG, 128)

    emb = _bag_sum_kernel()(tabs, idx_r).reshape(_B, _NT, _D)

    tw0x = tw0[:_D]
    w729 = jnp.zeros((_NI * _NI, tw0.shape[1]), jnp.float32).at[_PAIR_FLAT].set(tw0[_D:])
    out = _dense_forward(
        numerical_feature_batch, emb, bw0, bb0[None, :], bw1, bb1[None, :],
        bw2, bb2[None, :], tw0x, w729, tb0[None, :], tw1, tb1[None, :], tw2,
        tb2[None, :],
    )
    return out


# speedup vs baseline: 1.2193x; 1.1044x over previous
import functools

import jax
import jax.numpy as jnp
import numpy as np
from jax import lax
from jax.experimental import pallas as pl
from jax.experimental.pallas import tpu as pltpu
from jax.experimental.pallas import tpu_sc as plsc

_B = 4096
_DENSE = 13
_NT = 26
_VOCAB = 100000
_D = 64
_BAG = 20
_NI = _NT + 1

_NW = 32
_BAGS = _B * _NT
_BPW = _BAGS // _NW
_C = 32
_NCH = _BPW // _C
_IPC = _C * _BAG
_NG = _IPC // 128

_PAIR_ROWS = np.array([i for i in range(_NI) for _ in range(i)], dtype=np.int32)
_PAIR_COLS = np.array([j for i in range(_NI) for j in range(i)], dtype=np.int32)
_PAIR_FLAT = _PAIR_ROWS * _NI + _PAIR_COLS


def _sc_bag_sum_body(tables_hbm, idx_hbm, out_hbm, idx_v, rows_v, out_v,
                     sem_i0, sem_i1, sem_g0, sem_g1, sem_o0, sem_o1):
    w = lax.axis_index("s") * 2 + lax.axis_index("c")
    half = _NCH // 2

    def idx_start(c, b, sem):
        pltpu.async_copy(idx_hbm.at[w, c], idx_v.at[b], sem)

    def idx_wait(c, b, sem):
        pltpu.make_async_copy(idx_hbm.at[w, c], idx_v.at[b], sem).wait()

    def gather_start(b, sem):
        for j in range(_NG):
            pltpu.async_copy(
                tables_hbm.at[idx_v.at[b, j]],
                rows_v.at[b, pl.ds(j * 128, 128)], sem,
            )

    def gather_wait(b, sem):
        for j in range(_NG):
            pltpu.make_async_copy(
                tables_hbm.at[idx_v.at[b, j]],
                rows_v.at[b, pl.ds(j * 128, 128)], sem,
            ).wait()

    def out_start(c, b, sem):
        pltpu.async_copy(out_v.at[b], out_hbm.at[pl.ds(w * _BPW + c * _C, _C)], sem)

    def out_wait(b, sem):
        pltpu.make_async_copy(
            out_v.at[b], out_hbm.at[pl.ds(w * _BPW, _C)], sem
        ).wait()

    def reduce_chunk(b):
        def bag_body(i, carry2):
            base = i * _BAG
            for dd in range(_D // 16):
                sl = pl.ds(dd * 16, 16)
                acc = rows_v[b, base, sl]
                for j in range(1, _BAG):
                    acc = acc + rows_v[b, base + j, sl]
                out_v[b, i, sl] = acc
            return carry2

        lax.fori_loop(0, _C, bag_body, 0)

    idx_start(0, 0, sem_i0)
    idx_start(1, 1, sem_i1)
    idx_wait(0, 0, sem_i0)
    gather_start(0, sem_g0)

    def body(c2, carry):
        c = c2 * 2
        not_last = c2 < half - 1
        idx_wait(c + 1, 1, sem_i1)
        gather_start(1, sem_g1)
        gather_wait(0, sem_g0)
        pl.when(not_last)(lambda: idx_start(c + 2, 0, sem_i0))
        pl.when(c2 >= 1)(lambda: out_wait(0, sem_o0))
        reduce_chunk(0)
        out_start(c, 0, sem_o0)
        pl.when(not_last)(lambda: idx_wait(c + 2, 0, sem_i0))
        pl.when(not_last)(lambda: gather_start(0, sem_g0))
        gather_wait(1, sem_g1)
        pl.when(c2 >= 1)(lambda: out_wait(1, sem_o1))
        reduce_chunk(1)
        out_start(c + 1, 1, sem_o1)
        pl.when(not_last)(lambda: idx_start(c + 3, 1, sem_i1))
        return carry

    lax.fori_loop(0, half, body, 0)
    out_wait(0, sem_o0)
    out_wait(1, sem_o1)


@functools.cache
def _bag_sum_kernel():
    return pl.kernel(
        _sc_bag_sum_body,
        out_type=jax.ShapeDtypeStruct((_BAGS, _D), jnp.float32),
        mesh=plsc.VectorSubcoreMesh(core_axis_name="c", subcore_axis_name="s"),
        scratch_types=[
            pltpu.VMEM((2, _NG, 128), jnp.int32),
            pltpu.VMEM((2, _IPC, _D), jnp.float32),
            pltpu.VMEM((2, _C, _D), jnp.float32),
            pltpu.SemaphoreType.DMA,
            pltpu.SemaphoreType.DMA,
            pltpu.SemaphoreType.DMA,
            pltpu.SemaphoreType.DMA,
            pltpu.SemaphoreType.DMA,
            pltpu.SemaphoreType.DMA,
        ],
        compiler_params=pltpu.CompilerParams(use_tc_tiling_on_sc=False),
    )


def _dense_body(num_ref, emb_ref, bw0_ref, bb0_ref, bw1_ref, bb1_ref, bw2_ref,
                bb2_ref, tw0x_ref, w729_ref, tb0_ref, tw1_ref, tb1_ref,
                tw2_ref, tb2_ref, out_ref):
    f32 = jnp.float32
    x = num_ref[...]
    x = jnp.maximum(jnp.dot(x, bw0_ref[...], preferred_element_type=f32) + bb0_ref[...], 0.0)
    x = jnp.maximum(jnp.dot(x, bw1_ref[...], preferred_element_type=f32) + bb1_ref[...], 0.0)
    x = jnp.maximum(jnp.dot(x, bw2_ref[...], preferred_element_type=f32) + bb2_ref[...], 0.0)
    a = jnp.concatenate([x[:, None, :], emb_ref[...]], axis=1)
    zs = []
    for n in range(_NI):
        zs.append(jnp.sum(a * a[:, n : n + 1, :], axis=-1))
    z = jnp.concatenate(zs, axis=-1)
    t = jnp.dot(x, tw0x_ref[...], preferred_element_type=f32)
    t = t + jnp.dot(z, w729_ref[...], preferred_element_type=f32)
    t = jnp.maximum(t + tb0_ref[...], 0.0)
    t = jnp.maximum(jnp.dot(t, tw1_ref[...], preferred_element_type=f32) + tb1_ref[...], 0.0)
    out_ref[...] = jnp.maximum(
        jnp.dot(t, tw2_ref[...], preferred_element_type=f32) + tb2_ref[...], 0.0
    )


def _dense_forward(num, emb, bw0, bb0, bw1, bb1, bw2, bb2, tw0x, w729, tb0,
                   tw1, tb1, tw2, tb2):
    bb = 256
    grid = (_B // bb,)

    def row_spec(shape):
        return pl.BlockSpec((bb,) + shape[1:], lambda i: (i,) + (0,) * (len(shape) - 1))

    def full_spec(shape):
        return pl.BlockSpec(shape, lambda i: (0,) * len(shape))

    ws = [bw0, bb0, bw1, bb1, bw2, bb2, tw0x, w729, tb0, tw1, tb1, tw2, tb2]
    return pl.pallas_call(
        _dense_body,
        grid=grid,
        in_specs=[row_spec(num.shape), row_spec(emb.shape)] + [full_spec(w.shape) for w in ws],
        out_specs=row_spec((_B, 1)),
        out_shape=jax.ShapeDtypeStruct((_B, 1), jnp.float32),
    )(num, emb, *ws)


def kernel(numerical_feature_batch, embedding_index_batch_list,
           embedding_offset_batch_list, tables, bw0, bb0, bw1, bb1, bw2, bb2,
           tw0, tb0, tw1, tb1, tw2, tb2):
    del embedding_offset_batch_list
    tabs = tables.reshape(_NT * _VOCAB, _D)
    idx_bt = jnp.transpose(embedding_index_batch_list, (1, 0, 2))
    idx_bt = idx_bt + (jnp.arange(_NT, dtype=jnp.int32) * _VOCAB)[None, :, None]
    idx_r = idx_bt.reshape(_NW, _NCH, _NG, 128)

    emb = _bag_sum_kernel()(tabs, idx_r).reshape(_B, _NT, _D)

    tw0x = tw0[:_D]
    w729 = jnp.zeros((_NI * _NI, tw0.shape[1]), jnp.float32).at[_PAIR_FLAT].set(tw0[_D:])
    out = _dense_forward(
        numerical_feature_batch, emb, bw0, bb0[None, :], bw1, bb1[None, :],
        bw2, bb2[None, :], tw0x, w729, tb0[None, :], tw1, tb1[None, :], tw2,
        tb2[None, :],
    )
    return out


# speedup vs baseline: 1.4438x; 1.1841x over previous
import functools

import jax
import jax.numpy as jnp
import numpy as np
from jax import lax
from jax.experimental import pallas as pl
from jax.experimental.pallas import tpu as pltpu
from jax.experimental.pallas import tpu_sc as plsc

_B = 4096
_DENSE = 13
_NT = 26
_VOCAB = 100000
_D = 64
_BAG = 20
_NI = _NT + 1

_NW = 32
_BAGS = _B * _NT
_BPW = _BAGS // _NW
_C = 32
_NCH = _BPW // _C
_IPC = _C * _BAG
_NG = _IPC // 128

_PAIR_ROWS = np.array([i for i in range(_NI) for _ in range(i)], dtype=np.int32)
_PAIR_COLS = np.array([j for i in range(_NI) for j in range(i)], dtype=np.int32)
_PAIR_FLAT = _PAIR_ROWS * _NI + _PAIR_COLS


def _sc_bag_sum_body(tables_hbm, idx_hbm, out_hbm, idx_v, rows_v, out_v,
                     sem_i0, sem_i1, sem_g0, sem_g1, sem_o0, sem_o1):
    w = lax.axis_index("s") * 2 + lax.axis_index("c")
    half = _NCH // 2

    def idx_start(c, b, sem):
        pltpu.async_copy(idx_hbm.at[w, c], idx_v.at[b], sem)

    def idx_wait(c, b, sem):
        pltpu.make_async_copy(idx_hbm.at[w, c], idx_v.at[b], sem).wait()

    def gather_start(b, sem):
        for j in range(_NG):
            pltpu.async_copy(
                tables_hbm.at[idx_v.at[b, j]],
                rows_v.at[b, pl.ds(j * 128, 128)], sem,
            )

    def gather_wait(b, sem):
        for j in range(_NG):
            pltpu.make_async_copy(
                tables_hbm.at[idx_v.at[b, j]],
                rows_v.at[b, pl.ds(j * 128, 128)], sem,
            ).wait()

    def out_start(c, b, sem):
        pltpu.async_copy(out_v.at[b], out_hbm.at[pl.ds(w * _BPW + c * _C, _C)], sem)

    def out_wait(b, sem):
        pltpu.make_async_copy(
            out_v.at[b], out_hbm.at[pl.ds(w * _BPW, _C)], sem
        ).wait()

    def reduce_chunk(b):
        def bag_body(i, carry2):
            base = i * _BAG
            for dd in range(_D // 16):
                sl = pl.ds(dd * 16, 16)
                acc = rows_v[b, base, sl]
                for j in range(1, _BAG):
                    acc = acc + rows_v[b, base + j, sl]
                out_v[b, i, sl] = acc
            return carry2

        lax.fori_loop(0, _C, bag_body, 0)

    idx_start(0, 0, sem_i0)
    idx_start(1, 1, sem_i1)
    idx_wait(0, 0, sem_i0)
    gather_start(0, sem_g0)

    def body(c2, carry):
        c = c2 * 2
        not_last = c2 < half - 1
        idx_wait(c + 1, 1, sem_i1)
        gather_start(1, sem_g1)
        gather_wait(0, sem_g0)
        pl.when(not_last)(lambda: idx_start(c + 2, 0, sem_i0))
        pl.when(c2 >= 1)(lambda: out_wait(0, sem_o0))
        reduce_chunk(0)
        out_start(c, 0, sem_o0)
        pl.when(not_last)(lambda: idx_wait(c + 2, 0, sem_i0))
        pl.when(not_last)(lambda: gather_start(0, sem_g0))
        gather_wait(1, sem_g1)
        pl.when(c2 >= 1)(lambda: out_wait(1, sem_o1))
        reduce_chunk(1)
        out_start(c + 1, 1, sem_o1)
        pl.when(not_last)(lambda: idx_start(c + 3, 1, sem_i1))
        return carry

    lax.fori_loop(0, half, body, 0)
    out_wait(0, sem_o0)
    out_wait(1, sem_o1)


@functools.cache
def _bag_sum_kernel():
    return pl.kernel(
        _sc_bag_sum_body,
        out_type=jax.ShapeDtypeStruct((_BAGS, _D), jnp.float32),
        mesh=plsc.VectorSubcoreMesh(core_axis_name="c", subcore_axis_name="s"),
        scratch_types=[
            pltpu.VMEM((2, _NG, 128), jnp.int32),
            pltpu.VMEM((2, _IPC, _D), jnp.float32),
            pltpu.VMEM((2, _C, _D), jnp.float32),
            pltpu.SemaphoreType.DMA,
            pltpu.SemaphoreType.DMA,
            pltpu.SemaphoreType.DMA,
            pltpu.SemaphoreType.DMA,
            pltpu.SemaphoreType.DMA,
            pltpu.SemaphoreType.DMA,
        ],
        compiler_params=pltpu.CompilerParams(use_tc_tiling_on_sc=False),
    )


def _gram_body(num_ref, emb_ref, bw0_ref, bb0_ref, bw1_ref, bb1_ref, bw2_ref,
               bb2_ref, x_ref, zb_ref):
    f32 = jnp.float32
    bb = num_ref.shape[0]
    x = num_ref[...]
    x = jnp.maximum(jnp.dot(x, bw0_ref[...], preferred_element_type=f32) + bb0_ref[...], 0.0)
    x = jnp.maximum(jnp.dot(x, bw1_ref[...], preferred_element_type=f32) + bb1_ref[...], 0.0)
    x = jnp.maximum(jnp.dot(x, bw2_ref[...], preferred_element_type=f32) + bb2_ref[...], 0.0)
    x_ref[...] = x
    e = emb_ref[...]
    zpad = jnp.zeros((32 - _NI, _D), f32)
    for g in range(bb // 8):
        sg = jnp.concatenate(
            [
                jnp.concatenate([x[g * 8 + t][None, :], e[g * 8 + t], zpad], axis=0)
                for t in range(8)
            ],
            axis=0,
        )
        z4 = jax.lax.dot_general(
            sg, sg, (((1,), (1,)), ((), ())), preferred_element_type=f32
        )
        for t in range(8):
            zb_ref[(g * 8 + t) * 32 : (g * 8 + t + 1) * 32, :] = (
                z4[t * 32 : (t + 1) * 32, t * 32 : (t + 1) * 32]
            )


def _top_body(x_ref, z_ref, tw0x_ref, w1024_ref, tb0_ref, tw1_ref, tb1_ref,
              tw2_ref, tb2_ref, out_ref):
    f32 = jnp.float32
    t = jnp.dot(x_ref[...], tw0x_ref[...], preferred_element_type=f32)
    t = t + jnp.dot(z_ref[...], w1024_ref[...], preferred_element_type=f32)
    t = jnp.maximum(t + tb0_ref[...], 0.0)
    t = jnp.maximum(jnp.dot(t, tw1_ref[...], preferred_element_type=f32) + tb1_ref[...], 0.0)
    out_ref[...] = jnp.maximum(
        jnp.dot(t, tw2_ref[...], preferred_element_type=f32) + tb2_ref[...], 0.0
    )


def _dense_forward(num, emb, bw0, bb0, bw1, bb1, bw2, bb2, tw0x, w1024, tb0,
                   tw1, tb1, tw2, tb2):
    bb = 256
    grid = (_B // bb,)

    def row_spec(shape):
        return pl.BlockSpec((bb,) + shape[1:], lambda i: (i,) + (0,) * (len(shape) - 1))

    def blk_spec(shape):
        return pl.BlockSpec((bb * 32,) + shape[1:], lambda i: (i,) + (0,) * (len(shape) - 1))

    def full_spec(shape):
        return pl.BlockSpec(shape, lambda i: (0,) * len(shape))

    ws1 = [bw0, bb0, bw1, bb1, bw2, bb2]
    x, zb = pl.pallas_call(
        _gram_body,
        grid=grid,
        in_specs=[row_spec(num.shape), row_spec(emb.shape)] + [full_spec(w.shape) for w in ws1],
        out_specs=[row_spec((_B, _D)), blk_spec((_B * 32, 32))],
        out_shape=[
            jax.ShapeDtypeStruct((_B, _D), jnp.float32),
            jax.ShapeDtypeStruct((_B * 32, 32), jnp.float32),
        ],
    )(num, emb, *ws1)
    z = zb.reshape(_B, 1024)
    ws2 = [tw0x, w1024, tb0, tw1, tb1, tw2, tb2]
    return pl.pallas_call(
        _top_body,
        grid=grid,
        in_specs=[row_spec(x.shape), row_spec(z.shape)] + [full_spec(w.shape) for w in ws2],
        out_specs=row_spec((_B, 1)),
        out_shape=jax.ShapeDtypeStruct((_B, 1), jnp.float32),
    )(x, z, *ws2)


def kernel(numerical_feature_batch, embedding_index_batch_list,
           embedding_offset_batch_list, tables, bw0, bb0, bw1, bb1, bw2, bb2,
           tw0, tb0, tw1, tb1, tw2, tb2):
    del embedding_offset_batch_list
    tabs = tables.reshape(_NT * _VOCAB, _D)
    idx_bt = jnp.transpose(embedding_index_batch_list, (1, 0, 2))
    idx_bt = idx_bt + (jnp.arange(_NT, dtype=jnp.int32) * _VOCAB)[None, :, None]
    idx_r = idx_bt.reshape(_NW, _NCH, _NG, 128)

    emb = _bag_sum_kernel()(tabs, idx_r).reshape(_B, _NT, _D)

    tw0x = tw0[:_D]
    pair32 = _PAIR_ROWS * 32 + _PAIR_COLS
    w1024 = jnp.zeros((32 * 32, tw0.shape[1]), jnp.float32).at[pair32].set(tw0[_D:])
    out = _dense_forward(
        numerical_feature_batch, emb, bw0, bb0[None, :], bw1, bb1[None, :],
        bw2, bb2[None, :], tw0x, w1024, tb0[None, :], tw1, tb1[None, :], tw2,
        tb2[None, :],
    )
    return out
